# Initial kernel scaffold; baseline (speedup 1.0000x reference)
import jax
import jax.numpy as jnp
from jax.experimental import pallas as pl


def kernel(input_image_embeds, image_attention_mask):
    raise NotImplementedError("write your pallas kernel here")


# speedup vs baseline: 19.5985x; 19.5985x over previous
import functools

import jax
import jax.numpy as jnp
from jax import lax
from jax.experimental import pallas as pl
from jax.experimental.pallas import tpu as pltpu
from jax.experimental.pallas import tpu_sc as plsc

_B = 32
_N = 1024
_L = 16
_NUM_CORES = 2
_NUM_SUBCORES = 16

_MESH = plsc.VectorSubcoreMesh(core_axis_name="c", subcore_axis_name="s")


@functools.partial(
    pl.kernel,
    out_type=jax.ShapeDtypeStruct((_B, _N), jnp.int32),
    mesh=_MESH,
    scratch_types=[
        pltpu.VMEM((_N,), jnp.int32),
        pltpu.VMEM((_N,), jnp.int32),
    ],
    compiler_params=pltpu.CompilerParams(needs_layout_passes=False),
)
def _position_ids_sc(mask_hbm, out_hbm, mask_v, out_v):
    wid = lax.axis_index("s") * _NUM_CORES + lax.axis_index("c")
    pltpu.sync_copy(mask_hbm.at[wid], mask_v)

    def body(i, carry):
        v = mask_v[pl.ds(i * _L, _L)]
        incl = plsc.cumsum(v)
        rank = incl + (carry - 1)
        out_v[pl.ds(i * _L, _L)] = jnp.where(v > 0, rank, jnp.zeros_like(rank))
        return carry + jnp.sum(v)

    lax.fori_loop(0, _N // _L, body, jnp.int32(0))
    pltpu.sync_copy(out_v, out_hbm.at[wid])


def kernel(input_image_embeds, image_attention_mask):
    del input_image_embeds
    mask = image_attention_mask.reshape(_B, _N).astype(jnp.int32)
    return {"patch_position_ids": _position_ids_sc(mask)}
